# Initial kernel scaffold; baseline (speedup 1.0000x reference)
import jax
import jax.numpy as jnp
from jax.experimental import pallas as pl


def kernel(node_embeddings, batch_indices, W1, b1, W2, b2):
    raise NotImplementedError("write your pallas kernel here")


# speedup vs baseline: 4.7603x; 4.7603x over previous
import functools

import jax
import jax.numpy as jnp
from jax import lax
from jax.experimental import pallas as pl
from jax.experimental.pallas import tpu as pltpu
from jax.experimental.pallas import tpu_sc as plsc

N = 320000
D = 128
OUT = 128
G = 512

NC = 2
NS = 16
NW = NC * NS
RPW = N // NW
CH = 80
NCH = RPW // CH
GPT = G // NS

_mesh = plsc.VectorSubcoreMesh(
    core_axis_name="c", subcore_axis_name="s", num_cores=NC, num_subcores=NS
)


@functools.partial(
    pl.kernel,
    out_type=jax.ShapeDtypeStruct((NC, G, D), jnp.float32),
    mesh=_mesh,
    scratch_types=[
        pltpu.VMEM((NCH, CH), jnp.int32),
        pltpu.VMEM((CH, D), jnp.float32),
        pltpu.VMEM((GPT, D), jnp.float32),
        pltpu.VMEM_SHARED((G, D), jnp.float32),
    ],
)
def _segsum(rows_hbm, idx_hbm, out_hbm, idx_v, rows_v, zbuf_v, acc_sh):
    cid = lax.axis_index("c")
    sid = lax.axis_index("s")
    wid = cid * NS + sid
    base = wid * RPW

    zero16 = jnp.zeros((16,), jnp.float32)

    def zstep(i, carry):
        zbuf_v[i // (D // 16), pl.ds((i % (D // 16)) * 16, 16)] = zero16
        return carry

    lax.fori_loop(0, GPT * (D // 16), zstep, 0)
    pltpu.sync_copy(zbuf_v, acc_sh.at[pl.ds(sid * GPT, GPT)])
    plsc.subcore_barrier()

    pltpu.sync_copy(idx_hbm.at[wid], idx_v)

    def step(j, carry):
        pltpu.sync_copy(rows_hbm.at[pl.ds(base + j * CH, CH)], rows_v)
        pltpu.sync_copy(rows_v, acc_sh.at[idx_v.at[j]], add=True)
        return carry

    lax.fori_loop(0, NCH, step, 0)
    plsc.subcore_barrier()

    pltpu.sync_copy(
        acc_sh.at[pl.ds(sid * GPT, GPT)], out_hbm.at[cid, pl.ds(sid * GPT, GPT)]
    )


def _mlp_body(p_ref, w1_ref, b1_ref, w2_ref, b2_ref, o_ref):
    pooled = p_ref[0] + p_ref[1]
    h = jnp.maximum(
        jnp.dot(pooled, w1_ref[...], preferred_element_type=jnp.float32)
        + b1_ref[...],
        0.0,
    )
    o_ref[...] = (
        jnp.dot(h, w2_ref[...], preferred_element_type=jnp.float32) + b2_ref[...]
    )


def kernel(node_embeddings, batch_indices, W1, b1, W2, b2):
    idx32 = batch_indices.astype(jnp.int32).reshape(NW, NCH, CH)
    partial = _segsum(node_embeddings, idx32)
    return pl.pallas_call(
        _mlp_body,
        out_shape=jax.ShapeDtypeStruct((G, OUT), jnp.float32),
    )(partial, W1, b1.reshape(1, D), W2, b2.reshape(1, OUT))


# speedup vs baseline: 5.5354x; 1.1628x over previous
import functools

import jax
import jax.numpy as jnp
from jax import lax
from jax.experimental import pallas as pl
from jax.experimental.pallas import tpu as pltpu
from jax.experimental.pallas import tpu_sc as plsc

N = 320000
D = 128
OUT = 128
G = 512

NC = 2
NS = 16
NW = NC * NS
RPW = N // NW
CHB = 400
NITER = RPW // CHB
SUB = 5
CHS = CHB // SUB
GPT = G // NS

_mesh = plsc.VectorSubcoreMesh(
    core_axis_name="c", subcore_axis_name="s", num_cores=NC, num_subcores=NS
)


@functools.partial(
    pl.kernel,
    out_type=jax.ShapeDtypeStruct((NC, G, D), jnp.float32),
    mesh=_mesh,
    scratch_types=[
        pltpu.VMEM((2, SUB, CHS), jnp.int32),
        pltpu.VMEM((CHB, D), jnp.float32),
        pltpu.VMEM((CHB, D), jnp.float32),
        pltpu.VMEM_SHARED((G, D), jnp.float32),
        pltpu.SemaphoreType.DMA,
        pltpu.SemaphoreType.DMA,
        pltpu.SemaphoreType.DMA,
        pltpu.SemaphoreType.DMA,
    ],
)
def _segsum(rows_hbm, idx_hbm, out_hbm, idx_v, rows0, rows1,
            acc_sh, semg0, semg1, sems0, sems1):
    cid = lax.axis_index("c")
    sid = lax.axis_index("s")
    wid = cid * NS + sid
    base = wid * RPW
    bufs = (rows0, rows1)
    semg = (semg0, semg1)
    sems = (sems0, sems1)

    zero16 = jnp.zeros((16,), jnp.float32)

    def zstep(i, carry):
        rows0[i // (D // 16), pl.ds((i % (D // 16)) * 16, 16)] = zero16
        return carry

    lax.fori_loop(0, GPT * (D // 16), zstep, 0)
    pltpu.sync_copy(rows0.at[pl.ds(0, GPT)], acc_sh.at[pl.ds(sid * GPT, GPT)])

    gathers = [None, None]
    idxg = [None, None]
    scatters = [[], []]
    gathers[0] = pltpu.async_copy(rows_hbm.at[pl.ds(base, CHB)], bufs[0], semg[0])
    idxg[0] = pltpu.async_copy(idx_hbm.at[wid, 0], idx_v.at[0], semg[0])
    plsc.subcore_barrier()
    for j in range(NITER):
        b = j % 2
        gathers[b].wait()
        idxg[b].wait()
        for cp in scatters[1 - b]:
            cp.wait()
        scatters[1 - b] = []
        if j + 1 < NITER:
            gathers[1 - b] = pltpu.async_copy(
                rows_hbm.at[pl.ds(base + (j + 1) * CHB, CHB)],
                bufs[1 - b], semg[1 - b])
            idxg[1 - b] = pltpu.async_copy(
                idx_hbm.at[wid, j + 1], idx_v.at[1 - b], semg[1 - b])
        for k in range(SUB):
            scatters[b].append(pltpu.async_copy(
                bufs[b].at[pl.ds(k * CHS, CHS)],
                acc_sh.at[idx_v.at[b, k]], sems[b], add=True))
    for cp in scatters[(NITER - 1) % 2]:
        cp.wait()
    plsc.subcore_barrier()

    pltpu.sync_copy(
        acc_sh.at[pl.ds(sid * GPT, GPT)], out_hbm.at[cid, pl.ds(sid * GPT, GPT)]
    )


def _mlp_body(p_ref, w1_ref, b1_ref, w2_ref, b2_ref, o_ref):
    pooled = p_ref[0] + p_ref[1]
    h = jnp.maximum(
        jnp.dot(pooled, w1_ref[...], preferred_element_type=jnp.float32)
        + b1_ref[...],
        0.0,
    )
    o_ref[...] = (
        jnp.dot(h, w2_ref[...], preferred_element_type=jnp.float32) + b2_ref[...]
    )


def kernel(node_embeddings, batch_indices, W1, b1, W2, b2):
    idx32 = batch_indices.astype(jnp.int32).reshape(NW, NITER, SUB, CHS)
    partial = _segsum(node_embeddings, idx32)
    return pl.pallas_call(
        _mlp_body,
        out_shape=jax.ShapeDtypeStruct((G, OUT), jnp.float32),
    )(partial, W1, b1.reshape(1, D), W2, b2.reshape(1, OUT))
